# baseline (speedup 1.0000x reference)
import functools

import jax
import jax.numpy as jnp
from jax import lax
from jax.experimental import pallas as pl
from jax.experimental.pallas import tpu as pltpu
from jax.experimental.pallas import tpu_sc as plsc

VOCAB = 1000
VPAD = 1024
EMBED = 128
B = 1024
L = 50
BT = B * L

NUM_CORES = 2
NUM_SUBCORES = 16
NW = NUM_CORES * NUM_SUBCORES
B_PER_W = B // NW


def _gram_body(w_ref, wp_ref, b_ref, g_ref):
    w = w_ref[...]
    wp = wp_ref[...]
    g = lax.dot_general(
        w, wp,
        dimension_numbers=(((1,), (1,)), ((), ())),
        preferred_element_type=jnp.float32,
    )
    g_ref[...] = g + b_ref[...]


def _gram(W, Wp, b2d):
    return pl.pallas_call(
        _gram_body,
        out_shape=jax.ShapeDtypeStruct((VOCAB, b2d.shape[1]), jnp.float32),
    )(W, Wp, b2d)


_sc_mesh = plsc.VectorSubcoreMesh(core_axis_name="c", subcore_axis_name="s")


@functools.partial(
    pl.kernel,
    mesh=_sc_mesh,
    out_type=jax.ShapeDtypeStruct((B, L, VOCAB), jnp.float32),
    scratch_types=[
        pltpu.VMEM((B_PER_W, L), jnp.int32),
        pltpu.VMEM((L, VOCAB), jnp.float32),
        pltpu.SemaphoreType.DMA,
    ],
    compiler_params=pltpu.CompilerParams(use_tc_tiling_on_sc=False),
)
def _gather(table_hbm, idx_hbm, out_hbm, idx_v, buf_v, sem):
    wid = lax.axis_index("s") * NUM_CORES + lax.axis_index("c")
    base = wid * B_PER_W
    pltpu.sync_copy(idx_hbm.at[pl.ds(base, B_PER_W)], idx_v)

    def body(j, carry):
        pltpu.async_copy(table_hbm.at[idx_v.at[j]], buf_v, sem).wait()
        pltpu.sync_copy(buf_v, out_hbm.at[base + j])
        return carry

    lax.fori_loop(0, B_PER_W, body, 0)


def kernel(input_ids, W, b):
    ids = input_ids.astype(jnp.int32)
    table = _gram(W, W, b.reshape(1, VOCAB))
    return _gather(table, ids)


# speedup vs baseline: 1.4986x; 1.4986x over previous
import functools

import jax
import jax.numpy as jnp
from jax import lax
from jax.experimental import pallas as pl
from jax.experimental.pallas import tpu as pltpu
from jax.experimental.pallas import tpu_sc as plsc

VOCAB = 1000
VPAD = 1024
EMBED = 128
B = 1024
L = 50
BT = B * L

VMAIN = 896
VTAIL = VOCAB - VMAIN

NUM_CORES = 2
NUM_SUBCORES = 16
NW = NUM_CORES * NUM_SUBCORES
B_PER_W = B // NW


def _gram_body(w_ref, wp_ref, b_ref, g_ref):
    w = w_ref[...]
    wp = wp_ref[...]
    g = lax.dot_general(
        w, wp,
        dimension_numbers=(((1,), (1,)), ((), ())),
        preferred_element_type=jnp.float32,
    )
    g_ref[...] = g + b_ref[...]


def _gram(W, Wp, b2d):
    return pl.pallas_call(
        _gram_body,
        out_shape=jax.ShapeDtypeStruct((VOCAB, b2d.shape[1]), jnp.float32),
    )(W, Wp, b2d)


_sc_mesh = plsc.VectorSubcoreMesh(
    core_axis_name="c", subcore_axis_name="s",
    num_cores=NUM_CORES, num_subcores=NUM_SUBCORES)


L_MAIN = 48
L_TAIL = L - L_MAIN


def _gather_body(table_hbm, idx_hbm, out_hbm, idx_v, buf48, buf2, buf_tail,
                 sem):
    wid = lax.axis_index("s") * NUM_CORES + lax.axis_index("c")
    base = wid * B_PER_W
    pltpu.sync_copy(idx_hbm.at[pl.ds(base, B_PER_W)], idx_v)
    lanes = jnp.arange(16, dtype=jnp.int32)

    def slab(j, carry):
        pltpu.async_copy(
            table_hbm.at[idx_v.at[j, pl.ds(0, L_MAIN)]], buf48, sem).wait()
        pltpu.async_copy(
            table_hbm.at[idx_v.at[j, pl.ds(L_MAIN, L_TAIL)]], buf2,
            sem).wait()
        pltpu.sync_copy(buf48.at[:, pl.ds(0, VMAIN)],
                        out_hbm.at[base + j, pl.ds(0, L_MAIN), pl.ds(0, VMAIN)])
        pltpu.sync_copy(buf2.at[:, pl.ds(0, VMAIN)],
                        out_hbm.at[base + j, pl.ds(L_MAIN, L_TAIL),
                                   pl.ds(0, VMAIN)])
        def row(r, c2):
            for c in range(6):
                buf_tail[r, pl.ds(16 * c, 16)] = (
                    buf48[r, pl.ds(VMAIN + 16 * c, 16)])
            rr = jnp.full((16,), r, jnp.int32)
            x = plsc.load_gather(buf48, [rr, (VOCAB - 16) + lanes])
            plsc.store_scatter(buf_tail, [rr, (VTAIL - 16) + lanes], x)
            return c2
        lax.fori_loop(0, L_MAIN, row, 0)
        for k in range(L_TAIL):
            for c in range(6):
                buf_tail[L_MAIN + k, pl.ds(16 * c, 16)] = (
                    buf2[k, pl.ds(VMAIN + 16 * c, 16)])
            kk = jnp.full((16,), k, jnp.int32)
            rr = jnp.full((16,), L_MAIN + k, jnp.int32)
            x = plsc.load_gather(buf2, [kk, (VOCAB - 16) + lanes])
            plsc.store_scatter(buf_tail, [rr, (VTAIL - 16) + lanes], x)
        pltpu.sync_copy(buf_tail, out_hbm.at[base + j, :, pl.ds(VMAIN, VTAIL)])
        return carry

    lax.fori_loop(0, B_PER_W, slab, 0)


_gather = functools.partial(
    pl.kernel,
    mesh=_sc_mesh,
    out_type=jax.ShapeDtypeStruct((B, L, VOCAB), jnp.float32),
    scratch_types=[
        pltpu.VMEM((B_PER_W, L), jnp.int32),
        pltpu.VMEM((L_MAIN, VPAD), jnp.float32),
        pltpu.VMEM((L_TAIL, VPAD), jnp.float32),
        pltpu.VMEM((L, VTAIL), jnp.float32),
        pltpu.SemaphoreType.DMA,
    ],
    compiler_params=pltpu.CompilerParams(needs_layout_passes=False),
)(_gather_body)


def kernel(input_ids, W, b):
    ids = input_ids.astype(jnp.int32)
    Wp = jnp.zeros((VPAD, EMBED), jnp.float32).at[:VOCAB].set(W)
    bp = jnp.zeros((1, VPAD), jnp.float32).at[:, :VOCAB].set(b)
    table = _gram(W, Wp, bp)
    return _gather(table, ids)


# speedup vs baseline: 1.7543x; 1.1706x over previous
import functools

import jax
import jax.numpy as jnp
from jax import lax
from jax.experimental import pallas as pl
from jax.experimental.pallas import tpu as pltpu
from jax.experimental.pallas import tpu_sc as plsc

VOCAB = 1000
VPAD = 1024
EMBED = 128
B = 1024
L = 50
BT = B * L

VMAIN = 896
VTAIL = VOCAB - VMAIN

NUM_CORES = 2
NUM_SUBCORES = 16
NW = NUM_CORES * NUM_SUBCORES
B_PER_W = B // NW


def _gram_body(w_ref, wp_ref, b_ref, g_ref):
    w = w_ref[...]
    wp = wp_ref[...]
    g = lax.dot_general(
        w, wp,
        dimension_numbers=(((1,), (1,)), ((), ())),
        preferred_element_type=jnp.float32,
    )
    g_ref[...] = g + b_ref[...]


def _gram(W, Wp, b2d):
    return pl.pallas_call(
        _gram_body,
        out_shape=jax.ShapeDtypeStruct((VOCAB, b2d.shape[1]), jnp.float32),
    )(W, Wp, b2d)


_sc_mesh = plsc.VectorSubcoreMesh(
    core_axis_name="c", subcore_axis_name="s",
    num_cores=NUM_CORES, num_subcores=NUM_SUBCORES)


L_MAIN = 48
L_TAIL = L - L_MAIN


def _gather_body(table_hbm, idx_hbm, out_hbm, idx_v,
                 b48_0, b2_0, bt_0, b48_1, b2_1, bt_1,
                 sg0, sg1, sw0, sw1):
    wid = lax.axis_index("s") * NUM_CORES + lax.axis_index("c")
    base = wid * B_PER_W
    pltpu.sync_copy(idx_hbm.at[pl.ds(base, B_PER_W)], idx_v)
    lanes = jnp.arange(16, dtype=jnp.int32)

    slots = ((b48_0, b2_0, bt_0, sg0, sw0), (b48_1, b2_1, bt_1, sg1, sw1))

    def start_g(j, slot):
        b48, b2, _, sg, _ = slot
        pltpu.async_copy(table_hbm.at[idx_v.at[j, pl.ds(0, L_MAIN)]], b48, sg)
        pltpu.async_copy(
            table_hbm.at[idx_v.at[j, pl.ds(L_MAIN, L_TAIL)]], b2, sg)

    def wait_g(slot):
        b48, b2, _, sg, _ = slot
        pltpu.make_async_copy(table_hbm.at[pl.ds(0, L_MAIN)], b48, sg).wait()
        pltpu.make_async_copy(table_hbm.at[pl.ds(0, L_TAIL)], b2, sg).wait()

    def repack(slot):
        b48, b2, bt, _, _ = slot

        def row(r, c2):
            for c in range(6):
                bt[r, pl.ds(16 * c, 16)] = b48[r, pl.ds(VMAIN + 16 * c, 16)]
            rr = jnp.full((16,), r, jnp.int32)
            x = plsc.load_gather(b48, [rr, (VOCAB - 16) + lanes])
            plsc.store_scatter(bt, [rr, (VTAIL - 16) + lanes], x)
            return c2
        lax.fori_loop(0, L_MAIN, row, 0)
        for k in range(L_TAIL):
            for c in range(6):
                bt[L_MAIN + k, pl.ds(16 * c, 16)] = (
                    b2[k, pl.ds(VMAIN + 16 * c, 16)])
            kk = jnp.full((16,), k, jnp.int32)
            rr = jnp.full((16,), L_MAIN + k, jnp.int32)
            x = plsc.load_gather(b2, [kk, (VOCAB - 16) + lanes])
            plsc.store_scatter(bt, [rr, (VTAIL - 16) + lanes], x)

    def w_refs(j, slot):
        b48, b2, bt, _, sw = slot
        return (
            (b48.at[:, pl.ds(0, VMAIN)],
             out_hbm.at[base + j, pl.ds(0, L_MAIN), pl.ds(0, VMAIN)], sw),
            (b2.at[:, pl.ds(0, VMAIN)],
             out_hbm.at[base + j, pl.ds(L_MAIN, L_TAIL), pl.ds(0, VMAIN)], sw),
            (bt, out_hbm.at[base + j, :, pl.ds(VMAIN, VTAIL)], sw),
        )

    def start_w(j, slot):
        for src, dst, sw in w_refs(j, slot):
            pltpu.async_copy(src, dst, sw)

    def drain_w(slot):
        for src, dst, sw in w_refs(0, slot):
            pltpu.make_async_copy(src, dst, sw).wait()

    start_g(0, slots[0])
    start_g(1, slots[1])
    wait_g(slots[0])
    repack(slots[0])
    start_w(0, slots[0])

    def pair(jj, c):
        j1 = 2 * jj + 1
        drain_w(slots[0])
        start_g(j1 + 1, slots[0])
        wait_g(slots[1])
        repack(slots[1])
        start_w(j1, slots[1])
        j2 = 2 * jj + 2
        drain_w(slots[1])
        start_g(j2 + 1, slots[1])
        wait_g(slots[0])
        repack(slots[0])
        start_w(j2, slots[0])
        return c

    lax.fori_loop(0, (B_PER_W - 2) // 2, pair, 0)

    drain_w(slots[0])
    wait_g(slots[1])
    repack(slots[1])
    start_w(B_PER_W - 1, slots[1])
    drain_w(slots[1])


_gather = functools.partial(
    pl.kernel,
    mesh=_sc_mesh,
    out_type=jax.ShapeDtypeStruct((B, L, VOCAB), jnp.float32),
    scratch_types=[
        pltpu.VMEM((B_PER_W, L), jnp.int32),
        pltpu.VMEM((L_MAIN, VPAD), jnp.float32),
        pltpu.VMEM((L_TAIL, VPAD), jnp.float32),
        pltpu.VMEM((L, VTAIL), jnp.float32),
        pltpu.VMEM((L_MAIN, VPAD), jnp.float32),
        pltpu.VMEM((L_TAIL, VPAD), jnp.float32),
        pltpu.VMEM((L, VTAIL), jnp.float32),
        pltpu.SemaphoreType.DMA,
        pltpu.SemaphoreType.DMA,
        pltpu.SemaphoreType.DMA,
        pltpu.SemaphoreType.DMA,
    ],
    compiler_params=pltpu.CompilerParams(needs_layout_passes=False),
)(_gather_body)


def kernel(input_ids, W, b):
    ids = input_ids.astype(jnp.int32)
    Wp = jnp.zeros((VPAD, EMBED), jnp.float32).at[:VOCAB].set(W)
    bp = jnp.zeros((1, VPAD), jnp.float32).at[:, :VOCAB].set(b)
    table = _gram(W, Wp, bp)
    return _gather(table, ids)
